# baseline (speedup 1.0000x reference)
import functools

import jax
import jax.numpy as jnp
from jax.experimental import pallas as pl
from jax.experimental.pallas import tpu as pltpu


def _gat_body(atoms_ref, adjs_ref, e2_ref, w_ref, asrc_ref, adst_ref,
              wedge_ref, aedge_ref, bias_ref, out_ref, m_ref, *, n, e):
    b = pl.program_id(0)

    @pl.when(b == 0)
    def _build_m():
        vec = jnp.sum(wedge_ref[...] * aedge_ref[...], axis=1, keepdims=True)
        r = jax.lax.broadcasted_iota(jnp.int32, (n * e, n), 0)
        c = jax.lax.broadcasted_iota(jnp.int32, (n * e, n), 1)
        grp = (r // e) == c
        m = jnp.zeros((n * e, n), jnp.float32)
        for d in range(e):
            m = m + jnp.where(grp & ((r % e) == d), vec[d:d + 1, 0:1], 0.0)
        m_ref[...] = m.astype(jnp.bfloat16)

    x = atoms_ref[0]
    xl = jnp.dot(x.astype(jnp.bfloat16), w_ref[...].astype(jnp.bfloat16),
                 preferred_element_type=jnp.float32)
    a_src = jax.lax.dot_general(xl, asrc_ref[...], (((1,), (1,)), ((), ())),
                                preferred_element_type=jnp.float32)
    a_dst = jax.lax.dot_general(adst_ref[...], xl, (((1,), (1,)), ((), ())),
                                preferred_element_type=jnp.float32)
    eb = e2_ref[0].astype(jnp.bfloat16)
    a_edge = jnp.dot(eb, m_ref[...], preferred_element_type=jnp.float32)

    logits = a_src + a_dst + a_edge
    logits = jnp.where(logits >= 0, logits, 0.2 * logits)
    mask = adjs_ref[0] > 0.5
    ml = jnp.where(mask, logits, -1e9)
    mx = jnp.max(ml, axis=0, keepdims=True)
    ex = jnp.exp(ml - mx)
    s = jnp.sum(ex, axis=0, keepdims=True)
    att = jnp.where(mask, ex / s, 0.0)
    out = jax.lax.dot_general(att.astype(jnp.bfloat16), xl.astype(jnp.bfloat16),
                              (((0,), (0,)), ((), ())),
                              preferred_element_type=jnp.float32)
    out_ref[0] = out + bias_ref[...]


def kernel(atoms, adjs, edges, W, att_src, att_dst, W_edge, att_edge, bias):
    B, N, D = atoms.shape
    E = edges.shape[-1]
    C = W.shape[-1]
    e2 = edges.reshape(B, N, N * E)
    w2 = W.reshape(D, C)
    wedge = W_edge.reshape(E, C)
    bias2 = bias.reshape(1, C)

    body = functools.partial(_gat_body, n=N, e=E)
    out = pl.pallas_call(
        body,
        grid=(B,),
        in_specs=[
            pl.BlockSpec((1, N, D), lambda b: (b, 0, 0)),
            pl.BlockSpec((1, N, N), lambda b: (b, 0, 0)),
            pl.BlockSpec((1, N, N * E), lambda b: (b, 0, 0)),
            pl.BlockSpec((D, C), lambda b: (0, 0)),
            pl.BlockSpec((1, C), lambda b: (0, 0)),
            pl.BlockSpec((1, C), lambda b: (0, 0)),
            pl.BlockSpec((E, C), lambda b: (0, 0)),
            pl.BlockSpec((1, C), lambda b: (0, 0)),
            pl.BlockSpec((1, C), lambda b: (0, 0)),
        ],
        out_specs=pl.BlockSpec((1, N, C), lambda b: (b, 0, 0)),
        out_shape=jax.ShapeDtypeStruct((B, N, C), jnp.float32),
        scratch_shapes=[pltpu.VMEM((N * E, N), jnp.bfloat16)],
    )(atoms, adjs, e2, w2, att_src, att_dst, wedge, att_edge, bias2)
    return out
